# baseline (speedup 1.0000x reference)
import functools

import jax
import jax.numpy as jnp
from jax import lax
from jax.experimental import pallas as pl
from jax.experimental.pallas import tpu as pltpu
from jax.experimental.pallas import tpu_sc as plsc

VOCAB = 1000000
DIM = 64
BATCH = 16384

_NUM_CORES = 2
_NUM_SUBCORES = 16
_NW = _NUM_CORES * _NUM_SUBCORES
_BPW = BATCH // _NW
_HALF = _BPW // 2
_LANES = 16
_NGRP = _BPW // _LANES
_RING = 7


@functools.partial(
    pl.kernel,
    out_type=(
        jax.ShapeDtypeStruct((BATCH, 128), jnp.float32),
        jax.ShapeDtypeStruct((BATCH, 128), jnp.float32),
    ),
    mesh=plsc.VectorSubcoreMesh(core_axis_name="c", subcore_axis_name="s"),
    compiler_params=pltpu.CompilerParams(needs_layout_passes=False),
    scratch_types=[
        pltpu.VMEM((_BPW,), jnp.int32),
        pltpu.VMEM((_BPW,), jnp.int32),
        pltpu.VMEM((_LANES + _BPW,), jnp.int32),
        pltpu.VMEM((_LANES + _BPW,), jnp.int32),
        pltpu.VMEM((_BPW + _LANES,), jnp.int32),
        pltpu.VMEM((_BPW + _LANES,), jnp.int32),
        pltpu.VMEM((2 * _RING * DIM, 128), jnp.float32),
        pltpu.VMEM((_LANES, 128), jnp.float32),
        pltpu.VMEM((_LANES, 128), jnp.float32),
        pltpu.SemaphoreType.DMA,
        pltpu.SemaphoreType.DMA,
    ],
)
def _gather_sc(wit_hbm, wjt_hbm, si_hbm, sj_hbm, rows_i_hbm, rows_j_hbm,
               sidx_a, sidx_b, bstage_a, bstage_b, dlist_a, dlist_b,
               ring, rows_stage_a, rows_stage_b, sem_a, sem_b):
    wid = lax.axis_index("s") * _NUM_CORES + lax.axis_index("c")
    base = wid * _BPW
    lane = lax.iota(jnp.int32, _LANES)

    def dedup(s_hbm, sidx, bstage, dlist):
        pltpu.sync_copy(s_hbm.at[pl.ds(base, _BPW)], sidx)
        bstage[pl.ds(0, _LANES)] = jnp.full((_LANES,), -1, jnp.int32)

        def blk_body(g, carry):
            v = sidx[pl.ds(g * _LANES, _LANES)]
            bstage[pl.ds(_LANES + g * _LANES, _LANES)] = (
                lax.shift_right_logical(v, 7)
            )
            return carry

        lax.fori_loop(0, _NGRP, blk_body, 0)

        def dedup_body(g, cnt):
            cur = bstage[pl.ds(_LANES + g * _LANES, _LANES)]
            prev = bstage[pl.ds(_LANES - 1 + g * _LANES, _LANES)]
            m = cur != prev
            inc = plsc.cumsum(m.astype(jnp.int32))
            pos = cnt + inc - 1
            plsc.store_scatter(dlist, [pos], cur, mask=m)
            return cnt + inc[15]

        return lax.fori_loop(0, _NGRP, dedup_body, 0)

    cnt_a = dedup(si_hbm, sidx_a, bstage_a, dlist_a)
    cnt_b = dedup(sj_hbm, sidx_b, bstage_b, dlist_b)
    nb_a = lax.div(cnt_a + (_RING - 1), _RING)
    nb_b = lax.div(cnt_b + (_RING - 1), _RING)
    nb = lax.max(nb_a, nb_b)

    def fetch(wt_hbm, dlist, cnt, sem, t, soff):
        vec = dlist[pl.ds(t * _RING, _LANES)]
        for p in range(_RING):
            @pl.when(t * _RING + p < cnt)
            def _():
                blk = vec[p]
                pltpu.async_copy(
                    wt_hbm.at[:, pl.ds(blk * 128, 128)],
                    ring.at[pl.ds((soff + p) * DIM, DIM), :],
                    sem,
                )

    def drain(wt_hbm, cnt, sem, t):
        for p in range(_RING):
            @pl.when(t * _RING + p < cnt)
            def _():
                pltpu.make_async_copy(
                    wt_hbm.at[:, pl.ds(0, 128)],
                    ring.at[pl.ds(0, DIM), :],
                    sem,
                ).wait()

    def extract(sidx, bstage, dlist, cnt, rows_stage, rows_hbm, t, soff,
                cur_g):
        vec = dlist[pl.ds(t * _RING, _LANES)]
        valid = (t * _RING + lane < cnt) & (lane < _RING)
        bfirst = vec[0]
        blast = lax.reduce_max(
            jnp.where(valid, vec, jnp.full((_LANES,), -1, jnp.int32)),
            axes=(0,),
        )
        safe_blk = [
            jnp.where(t * _RING + p < cnt, vec[p], -2 - p)
            for p in range(_RING)
        ]

        def grp_body(g, nflushed):
            bv = bstage[pl.ds(_LANES + g * _LANES, _LANES)]

            @pl.when((bv[15] >= bfirst) & (bv[0] <= blast))
            def _():
                cols = lax.bitwise_and(sidx[pl.ds(g * _LANES, _LANES)], 127)
                slot = jnp.zeros((_LANES,), jnp.int32)
                for p in range(_RING):
                    slot = slot + jnp.where(
                        bv == jnp.full((_LANES,), safe_blk[p], jnp.int32),
                        p + 1,
                        0,
                    )
                for l in range(_LANES):
                    s_l = slot[l] - 1

                    @pl.when(s_l >= 0)
                    def _():
                        rbase = (soff + s_l) * DIM
                        csp = jnp.full((_LANES,), cols[l], jnp.int32)
                        for f in range(DIM // _LANES):
                            vfeat = plsc.load_gather(
                                ring, [rbase + f * _LANES + lane, csp]
                            )
                            rows_stage[l, pl.ds(f * _LANES, _LANES)] = vfeat

                @pl.when((bv[15] >= bfirst) & (bv[15] <= blast))
                def _():
                    pltpu.sync_copy(
                        rows_stage,
                        rows_hbm.at[pl.ds(base + g * _LANES, _LANES), :],
                    )

            return nflushed + jnp.where(
                (bv[15] >= bfirst) & (bv[15] <= blast), 1, 0
            )

        nfl = lax.fori_loop(cur_g, _NGRP, grp_body, 0)
        return cur_g + nfl

    fetch(wit_hbm, dlist_a, cnt_a, sem_a, 0, 0)
    fetch(wjt_hbm, dlist_b, cnt_b, sem_b, 0, _RING)

    def batch_body(t, carry):
        cg_a, cg_b = carry

        @pl.when(t < nb_a)
        def _():
            drain(wit_hbm, cnt_a, sem_a, t)

        def do_a(_):
            return extract(sidx_a, bstage_a, dlist_a, cnt_a, rows_stage_a,
                           rows_i_hbm, t, 0, cg_a)

        cg_a2 = lax.cond(t < nb_a, do_a, lambda _: cg_a, 0)

        @pl.when(t + 1 < nb_a)
        def _():
            fetch(wit_hbm, dlist_a, cnt_a, sem_a, t + 1, 0)

        @pl.when(t < nb_b)
        def _():
            drain(wjt_hbm, cnt_b, sem_b, t)

        def do_b(_):
            return extract(sidx_b, bstage_b, dlist_b, cnt_b, rows_stage_b,
                           rows_j_hbm, t, _RING, cg_b)

        cg_b2 = lax.cond(t < nb_b, do_b, lambda _: cg_b, 0)

        @pl.when(t + 1 < nb_b)
        def _():
            fetch(wjt_hbm, dlist_b, cnt_b, sem_b, t + 1, _RING)

        return (cg_a2, cg_b2)

    lax.fori_loop(0, nb, batch_body, (0, 0))


@functools.partial(
    pl.kernel,
    out_type=jax.ShapeDtypeStruct((BATCH,), jnp.float32),
    mesh=plsc.VectorSubcoreMesh(core_axis_name="c", subcore_axis_name="s"),
    compiler_params=pltpu.CompilerParams(
        needs_layout_passes=False, use_tc_tiling_on_sc=False
    ),
    scratch_types=[
        pltpu.VMEM((_BPW,), jnp.int32),
        pltpu.VMEM((_BPW,), jnp.int32),
        pltpu.VMEM((_HALF,), jnp.int32),
        pltpu.VMEM((_HALF, 128), jnp.float32),
        pltpu.VMEM((_HALF, 128), jnp.float32),
        pltpu.VMEM((_BPW,), jnp.float32),
        pltpu.VMEM((_BPW,), jnp.float32),
        pltpu.VMEM((_BPW,), jnp.float32),
        pltpu.SemaphoreType.DMA,
        pltpu.SemaphoreType.DMA,
    ],
)
def _dot_sc(rows_i_hbm, rows_j_hbm, bi_hbm, bj_hbm, ii_hbm, jj_hbm,
            ri_hbm, rj_hbm, out_hbm,
            idx_v, rank_v, gidx, buf_a, buf_b, bias_v, bias_t, out_v,
            sem, sem2):
    wid = lax.axis_index("s") * _NUM_CORES + lax.axis_index("c")
    base = wid * _BPW
    lane = lax.iota(jnp.int32, _LANES)

    pltpu.sync_copy(ii_hbm.at[pl.ds(base, _BPW)], idx_v)
    pltpu.sync_copy(jj_hbm.at[pl.ds(base, _BPW)], rank_v)
    cpb_i = pltpu.async_copy(bi_hbm.at[idx_v], bias_v, sem)
    cpb_j = pltpu.async_copy(bj_hbm.at[rank_v], bias_t, sem2)
    cpb_i.wait()
    cpb_j.wait()

    def add_body(g, carry):
        rb = g * _LANES
        bias_v[pl.ds(rb, _LANES)] = (
            bias_v[pl.ds(rb, _LANES)] + bias_t[pl.ds(rb, _LANES)]
        )
        return carry

    lax.fori_loop(0, _NGRP, add_body, 0)

    def half_body(h):
        pltpu.sync_copy(ri_hbm.at[pl.ds(base + h * _HALF, _HALF)], gidx)
        pltpu.sync_copy(rj_hbm.at[pl.ds(base + h * _HALF, _HALF)], rank_v.at[pl.ds(0, _HALF)])
        cp_a = pltpu.async_copy(rows_i_hbm.at[gidx], buf_a, sem)
        cp_b = pltpu.async_copy(
            rows_j_hbm.at[rank_v.at[pl.ds(0, _HALF)]], buf_b, sem2
        )
        cp_a.wait()
        cp_b.wait()

        def body(g, carry):
            rb = g * _LANES
            acc = bias_v[pl.ds(h * _HALF + rb, _LANES)]
            rows = rb + lane
            for d in range(DIM):
                col = jnp.full((_LANES,), d, jnp.int32)
                a = plsc.load_gather(buf_a, [rows, col])
                b = plsc.load_gather(buf_b, [rows, col])
                acc = acc + a * b
            out_v[pl.ds(h * _HALF + rb, _LANES)] = acc
            return carry

        lax.fori_loop(0, _HALF // _LANES, body, 0)

    half_body(0)
    half_body(1)

    pltpu.sync_copy(out_v, out_hbm.at[pl.ds(base, _BPW)])


def kernel(i_indices, j_indices, wi, wj, bi, bj):
    ii = i_indices.astype(jnp.int32)
    jj = j_indices.astype(jnp.int32)
    arange = jnp.arange(BATCH, dtype=jnp.int32)
    pi = jnp.argsort(ii)
    si = ii[pi]
    rank_i = jnp.zeros((BATCH,), jnp.int32).at[pi].set(arange)
    pj = jnp.argsort(jj)
    sj = jj[pj]
    rank_j = jnp.zeros((BATCH,), jnp.int32).at[pj].set(arange)

    rows_i, rows_j = _gather_sc(wi.T, wj.T, si, sj)
    return _dot_sc(
        rows_i,
        rows_j,
        bi.reshape(VOCAB),
        bj.reshape(VOCAB),
        ii,
        jj,
        rank_i,
        rank_j,
    )


# speedup vs baseline: 1.0046x; 1.0046x over previous
import functools

import jax
import jax.numpy as jnp
from jax import lax
from jax.experimental import pallas as pl
from jax.experimental.pallas import tpu as pltpu
from jax.experimental.pallas import tpu_sc as plsc

VOCAB = 1000000
DIM = 64
BATCH = 16384

_NUM_CORES = 2
_NUM_SUBCORES = 16
_NW = _NUM_CORES * _NUM_SUBCORES
_BPW = BATCH // _NW
_HALF = _BPW // 2
_LANES = 16
_NGRP = _BPW // _LANES
_RING = 7


@functools.partial(
    pl.kernel,
    out_type=(
        jax.ShapeDtypeStruct((BATCH, 128), jnp.float32),
        jax.ShapeDtypeStruct((BATCH, 128), jnp.float32),
    ),
    mesh=plsc.VectorSubcoreMesh(core_axis_name="c", subcore_axis_name="s"),
    compiler_params=pltpu.CompilerParams(needs_layout_passes=False),
    scratch_types=[
        pltpu.VMEM((_BPW,), jnp.int32),
        pltpu.VMEM((_LANES + _BPW,), jnp.int32),
        pltpu.VMEM((_BPW + _LANES,), jnp.int32),
        pltpu.VMEM((2 * _RING * DIM, 128), jnp.float32),
        pltpu.VMEM((_LANES, 128), jnp.float32),
        pltpu.SemaphoreType.DMA,
    ],
)
def _gather_sc(wit_hbm, wjt_hbm, si_hbm, sj_hbm, rows_i_hbm, rows_j_hbm,
               sidx, bstage, dlist, ring, rows_stage, sem):
    wid = lax.axis_index("s") * _NUM_CORES + lax.axis_index("c")
    base = wid * _BPW
    lane = lax.iota(jnp.int32, _LANES)

    def run_table(wt_hbm, s_hbm, rows_hbm):
        pltpu.sync_copy(s_hbm.at[pl.ds(base, _BPW)], sidx)

        bstage[pl.ds(0, _LANES)] = jnp.full((_LANES,), -1, jnp.int32)

        def blk_body(g, carry):
            v = sidx[pl.ds(g * _LANES, _LANES)]
            bstage[pl.ds(_LANES + g * _LANES, _LANES)] = (
                lax.shift_right_logical(v, 7)
            )
            return carry

        lax.fori_loop(0, _NGRP, blk_body, 0)

        def dedup_body(g, cnt):
            cur = bstage[pl.ds(_LANES + g * _LANES, _LANES)]
            prev = bstage[pl.ds(_LANES - 1 + g * _LANES, _LANES)]
            m = cur != prev
            inc = plsc.cumsum(m.astype(jnp.int32))
            pos = cnt + inc - 1
            plsc.store_scatter(dlist, [pos], cur, mask=m)
            return cnt + inc[15]

        cnt = lax.fori_loop(0, _NGRP, dedup_body, 0)
        nb = lax.div(cnt + (_RING - 1), _RING)

        def fetch(t, half):
            vec = dlist[pl.ds(t * _RING, _LANES)]
            for p in range(_RING):
                @pl.when(t * _RING + p < cnt)
                def _():
                    blk = vec[p]
                    pltpu.async_copy(
                        wt_hbm.at[:, pl.ds(blk * 128, 128)],
                        ring.at[pl.ds((half * _RING + p) * DIM, DIM), :],
                        sem,
                    )

        def drain(t):
            for p in range(_RING):
                @pl.when(t * _RING + p < cnt)
                def _():
                    pltpu.make_async_copy(
                        wt_hbm.at[:, pl.ds(0, 128)],
                        ring.at[pl.ds(0, DIM), :],
                        sem,
                    ).wait()

        fetch(0, 0)

        def batch_body(t, cur_g):
            half = lax.rem(t, 2)
            drain(t)

            @pl.when(t + 1 < nb)
            def _():
                fetch(t + 1, lax.rem(t + 1, 2))

            vec = dlist[pl.ds(t * _RING, _LANES)]
            valid = (t * _RING + lane < cnt) & (lane < _RING)
            bfirst = vec[0]
            blast = lax.reduce_max(
                jnp.where(valid, vec, jnp.full((_LANES,), -1, jnp.int32)),
                axes=(0,),
            )
            safe_blk = [
                jnp.where(t * _RING + p < cnt, vec[p], -2 - p)
                for p in range(_RING)
            ]

            def grp_body(g, nflushed):
                bv = bstage[pl.ds(_LANES + g * _LANES, _LANES)]

                @pl.when((bv[15] >= bfirst) & (bv[0] <= blast))
                def _():
                    cols = lax.bitwise_and(
                        sidx[pl.ds(g * _LANES, _LANES)], 127
                    )
                    slot = jnp.zeros((_LANES,), jnp.int32)
                    for p in range(_RING):
                        slot = slot + jnp.where(
                            bv == jnp.full((_LANES,), safe_blk[p], jnp.int32),
                            p + 1,
                            0,
                        )
                    for l in range(_LANES):
                        s_l = slot[l] - 1

                        @pl.when(s_l >= 0)
                        def _():
                            rbase = (half * _RING + s_l) * DIM
                            csp = jnp.full((_LANES,), cols[l], jnp.int32)
                            for f in range(DIM // _LANES):
                                vfeat = plsc.load_gather(
                                    ring, [rbase + f * _LANES + lane, csp]
                                )
                                rows_stage[l, pl.ds(f * _LANES, _LANES)] = (
                                    vfeat
                                )

                    @pl.when((bv[15] >= bfirst) & (bv[15] <= blast))
                    def _():
                        pltpu.sync_copy(
                            rows_stage,
                            rows_hbm.at[pl.ds(base + g * _LANES, _LANES), :],
                        )

                return nflushed + jnp.where(
                    (bv[15] >= bfirst) & (bv[15] <= blast), 1, 0
                )

            nfl = lax.fori_loop(cur_g, _NGRP, grp_body, 0)
            return cur_g + nfl

        lax.fori_loop(0, nb, batch_body, 0)

    run_table(wit_hbm, si_hbm, rows_i_hbm)
    run_table(wjt_hbm, sj_hbm, rows_j_hbm)


@functools.partial(
    pl.kernel,
    out_type=jax.ShapeDtypeStruct((BATCH,), jnp.float32),
    mesh=plsc.VectorSubcoreMesh(core_axis_name="c", subcore_axis_name="s"),
    compiler_params=pltpu.CompilerParams(
        needs_layout_passes=False, use_tc_tiling_on_sc=False
    ),
    scratch_types=[
        pltpu.VMEM((_BPW,), jnp.int32),
        pltpu.VMEM((_BPW,), jnp.int32),
        pltpu.VMEM((_HALF,), jnp.int32),
        pltpu.VMEM((_HALF, 128), jnp.float32),
        pltpu.VMEM((_HALF, 128), jnp.float32),
        pltpu.VMEM((_BPW,), jnp.float32),
        pltpu.VMEM((_BPW,), jnp.float32),
        pltpu.VMEM((_BPW,), jnp.float32),
        pltpu.SemaphoreType.DMA,
        pltpu.SemaphoreType.DMA,
    ],
)
def _dot_sc(rows_i_hbm, rows_j_hbm, bi_hbm, bj_hbm, ii_hbm, jj_hbm,
            ri_hbm, rj_hbm, out_hbm,
            idx_v, rank_v, gidx, buf_a, buf_b, bias_v, bias_t, out_v,
            sem, sem2):
    wid = lax.axis_index("s") * _NUM_CORES + lax.axis_index("c")
    base = wid * _BPW
    lane = lax.iota(jnp.int32, _LANES)

    pltpu.sync_copy(ii_hbm.at[pl.ds(base, _BPW)], idx_v)
    pltpu.sync_copy(jj_hbm.at[pl.ds(base, _BPW)], rank_v)
    cpb_i = pltpu.async_copy(bi_hbm.at[idx_v], bias_v, sem)
    cpb_j = pltpu.async_copy(bj_hbm.at[rank_v], bias_t, sem2)
    cpb_i.wait()
    cpb_j.wait()

    def add_body(g, carry):
        rb = g * _LANES
        bias_v[pl.ds(rb, _LANES)] = (
            bias_v[pl.ds(rb, _LANES)] + bias_t[pl.ds(rb, _LANES)]
        )
        return carry

    lax.fori_loop(0, _NGRP, add_body, 0)

    def half_body(h):
        pltpu.sync_copy(ri_hbm.at[pl.ds(base + h * _HALF, _HALF)], gidx)
        pltpu.sync_copy(rj_hbm.at[pl.ds(base + h * _HALF, _HALF)], rank_v.at[pl.ds(0, _HALF)])
        cp_a = pltpu.async_copy(rows_i_hbm.at[gidx], buf_a, sem)
        cp_b = pltpu.async_copy(
            rows_j_hbm.at[rank_v.at[pl.ds(0, _HALF)]], buf_b, sem2
        )
        cp_a.wait()
        cp_b.wait()

        def body(g, carry):
            rb = g * _LANES
            acc = bias_v[pl.ds(h * _HALF + rb, _LANES)]
            rows = rb + lane
            for d in range(DIM):
                col = jnp.full((_LANES,), d, jnp.int32)
                a = plsc.load_gather(buf_a, [rows, col])
                b = plsc.load_gather(buf_b, [rows, col])
                acc = acc + a * b
            out_v[pl.ds(h * _HALF + rb, _LANES)] = acc
            return carry

        lax.fori_loop(0, _HALF // _LANES, body, 0)

    half_body(0)
    half_body(1)

    pltpu.sync_copy(out_v, out_hbm.at[pl.ds(base, _BPW)])


def kernel(i_indices, j_indices, wi, wj, bi, bj):
    ii = i_indices.astype(jnp.int32)
    jj = j_indices.astype(jnp.int32)
    arange = jnp.arange(BATCH, dtype=jnp.int32)
    pi = jnp.argsort(ii)
    si = ii[pi]
    rank_i = jnp.zeros((BATCH,), jnp.int32).at[pi].set(arange)
    pj = jnp.argsort(jj)
    sj = jj[pj]
    rank_j = jnp.zeros((BATCH,), jnp.int32).at[pj].set(arange)

    rows_i, rows_j = _gather_sc(wi.T, wj.T, si, sj)
    return _dot_sc(
        rows_i,
        rows_j,
        bi.reshape(VOCAB),
        bj.reshape(VOCAB),
        ii,
        jj,
        rank_i,
        rank_j,
    )


# speedup vs baseline: 1.0049x; 1.0003x over previous
import functools

import jax
import jax.numpy as jnp
from jax import lax
from jax.experimental import pallas as pl
from jax.experimental.pallas import tpu as pltpu
from jax.experimental.pallas import tpu_sc as plsc

VOCAB = 1000000
DIM = 64
BATCH = 16384

_NUM_CORES = 2
_NUM_SUBCORES = 16
_NW = _NUM_CORES * _NUM_SUBCORES
_BPW = BATCH // _NW
_HALF = _BPW // 2
_LANES = 16
_NGRP = _BPW // _LANES
_RING = 7


@functools.partial(
    pl.kernel,
    out_type=(
        jax.ShapeDtypeStruct((BATCH, 128), jnp.float32),
        jax.ShapeDtypeStruct((BATCH, 128), jnp.float32),
    ),
    mesh=plsc.VectorSubcoreMesh(core_axis_name="c", subcore_axis_name="s"),
    compiler_params=pltpu.CompilerParams(needs_layout_passes=False),
    scratch_types=[
        pltpu.VMEM((_BPW,), jnp.int32),
        pltpu.VMEM((_LANES + _BPW,), jnp.int32),
        pltpu.VMEM((_BPW + _LANES,), jnp.int32),
        pltpu.VMEM((2 * _RING * DIM, 128), jnp.float32),
        pltpu.VMEM((_LANES, 128), jnp.float32),
        pltpu.SemaphoreType.DMA,
    ],
)
def _gather_sc(wit_hbm, wjt_hbm, si_hbm, sj_hbm, rows_i_hbm, rows_j_hbm,
               sidx, bstage, dlist, ring, rows_stage, sem):
    wid = lax.axis_index("s") * _NUM_CORES + lax.axis_index("c")
    base = wid * _BPW
    lane = lax.iota(jnp.int32, _LANES)

    def run_table(wt_hbm, s_hbm, rows_hbm):
        pltpu.sync_copy(s_hbm.at[pl.ds(base, _BPW)], sidx)

        bstage[pl.ds(0, _LANES)] = jnp.full((_LANES,), -1, jnp.int32)

        def blk_body(g, carry):
            v = sidx[pl.ds(g * _LANES, _LANES)]
            bstage[pl.ds(_LANES + g * _LANES, _LANES)] = (
                lax.shift_right_logical(v, 7)
            )
            return carry

        lax.fori_loop(0, _NGRP, blk_body, 0)

        def dedup_body(g, cnt):
            cur = bstage[pl.ds(_LANES + g * _LANES, _LANES)]
            prev = bstage[pl.ds(_LANES - 1 + g * _LANES, _LANES)]
            m = cur != prev
            inc = plsc.cumsum(m.astype(jnp.int32))
            pos = cnt + inc - 1
            plsc.store_scatter(dlist, [pos], cur, mask=m)
            return cnt + inc[15]

        cnt = lax.fori_loop(0, _NGRP, dedup_body, 0)
        nb = lax.div(cnt + (_RING - 1), _RING)

        def fetch(t, half):
            vec = dlist[pl.ds(t * _RING, _LANES)]
            for p in range(_RING):
                @pl.when(t * _RING + p < cnt)
                def _():
                    blk = vec[p]
                    pltpu.async_copy(
                        wt_hbm.at[:, pl.ds(blk * 128, 128)],
                        ring.at[pl.ds((half * _RING + p) * DIM, DIM), :],
                        sem,
                    )

        def drain(t):
            for p in range(_RING):
                @pl.when(t * _RING + p < cnt)
                def _():
                    pltpu.make_async_copy(
                        wt_hbm.at[:, pl.ds(0, 128)],
                        ring.at[pl.ds(0, DIM), :],
                        sem,
                    ).wait()

        fetch(0, 0)

        def batch_body(t, cur_g):
            half = lax.rem(t, 2)
            drain(t)

            @pl.when(t + 1 < nb)
            def _():
                fetch(t + 1, lax.rem(t + 1, 2))

            vec = dlist[pl.ds(t * _RING, _LANES)]
            valid = (t * _RING + lane < cnt) & (lane < _RING)
            bfirst = vec[0]
            blast = lax.reduce_max(
                jnp.where(valid, vec, jnp.full((_LANES,), -1, jnp.int32)),
                axes=(0,),
            )
            safe_blk = [
                jnp.where(t * _RING + p < cnt, vec[p], -2 - p)
                for p in range(_RING)
            ]

            def grp_body(g, nflushed):
                bv = bstage[pl.ds(_LANES + g * _LANES, _LANES)]

                @pl.when((bv[15] >= bfirst) & (bv[0] <= blast))
                def _():
                    cols = lax.bitwise_and(
                        sidx[pl.ds(g * _LANES, _LANES)], 127
                    )
                    slot = jnp.zeros((_LANES,), jnp.int32)
                    for p in range(_RING):
                        slot = slot + jnp.where(
                            bv == jnp.full((_LANES,), safe_blk[p], jnp.int32),
                            p + 1,
                            0,
                        )
                    m = slot > 0
                    rbase_v = (
                        half * _RING + jnp.maximum(slot - 1, 0)
                    ) * DIM
                    for d in range(DIM):
                        dsp = jnp.full((_LANES,), d, jnp.int32)
                        vals = plsc.load_gather(
                            ring, [rbase_v + d, cols], mask=m
                        )
                        plsc.store_scatter(
                            rows_stage, [lane, dsp], vals, mask=m
                        )

                    @pl.when((bv[15] >= bfirst) & (bv[15] <= blast))
                    def _():
                        pltpu.sync_copy(
                            rows_stage,
                            rows_hbm.at[pl.ds(base + g * _LANES, _LANES), :],
                        )

                return nflushed + jnp.where(
                    (bv[15] >= bfirst) & (bv[15] <= blast), 1, 0
                )

            nfl = lax.fori_loop(cur_g, _NGRP, grp_body, 0)
            return cur_g + nfl

        lax.fori_loop(0, nb, batch_body, 0)

    run_table(wit_hbm, si_hbm, rows_i_hbm)
    run_table(wjt_hbm, sj_hbm, rows_j_hbm)


@functools.partial(
    pl.kernel,
    out_type=jax.ShapeDtypeStruct((BATCH,), jnp.float32),
    mesh=plsc.VectorSubcoreMesh(core_axis_name="c", subcore_axis_name="s"),
    compiler_params=pltpu.CompilerParams(
        needs_layout_passes=False, use_tc_tiling_on_sc=False
    ),
    scratch_types=[
        pltpu.VMEM((_BPW,), jnp.int32),
        pltpu.VMEM((_BPW,), jnp.int32),
        pltpu.VMEM((_HALF,), jnp.int32),
        pltpu.VMEM((_HALF, 128), jnp.float32),
        pltpu.VMEM((_HALF, 128), jnp.float32),
        pltpu.VMEM((_BPW,), jnp.float32),
        pltpu.VMEM((_BPW,), jnp.float32),
        pltpu.VMEM((_BPW,), jnp.float32),
        pltpu.SemaphoreType.DMA,
        pltpu.SemaphoreType.DMA,
    ],
)
def _dot_sc(rows_i_hbm, rows_j_hbm, bi_hbm, bj_hbm, ii_hbm, jj_hbm,
            ri_hbm, rj_hbm, out_hbm,
            idx_v, rank_v, gidx, buf_a, buf_b, bias_v, bias_t, out_v,
            sem, sem2):
    wid = lax.axis_index("s") * _NUM_CORES + lax.axis_index("c")
    base = wid * _BPW
    lane = lax.iota(jnp.int32, _LANES)

    pltpu.sync_copy(ii_hbm.at[pl.ds(base, _BPW)], idx_v)
    pltpu.sync_copy(jj_hbm.at[pl.ds(base, _BPW)], rank_v)
    cpb_i = pltpu.async_copy(bi_hbm.at[idx_v], bias_v, sem)
    cpb_j = pltpu.async_copy(bj_hbm.at[rank_v], bias_t, sem2)
    cpb_i.wait()
    cpb_j.wait()

    def add_body(g, carry):
        rb = g * _LANES
        bias_v[pl.ds(rb, _LANES)] = (
            bias_v[pl.ds(rb, _LANES)] + bias_t[pl.ds(rb, _LANES)]
        )
        return carry

    lax.fori_loop(0, _NGRP, add_body, 0)

    def half_body(h):
        pltpu.sync_copy(ri_hbm.at[pl.ds(base + h * _HALF, _HALF)], gidx)
        pltpu.sync_copy(rj_hbm.at[pl.ds(base + h * _HALF, _HALF)], rank_v.at[pl.ds(0, _HALF)])
        cp_a = pltpu.async_copy(rows_i_hbm.at[gidx], buf_a, sem)
        cp_b = pltpu.async_copy(
            rows_j_hbm.at[rank_v.at[pl.ds(0, _HALF)]], buf_b, sem2
        )
        cp_a.wait()
        cp_b.wait()

        def body(g, carry):
            rb = g * _LANES
            acc = bias_v[pl.ds(h * _HALF + rb, _LANES)]
            rows = rb + lane
            for d in range(DIM):
                col = jnp.full((_LANES,), d, jnp.int32)
                a = plsc.load_gather(buf_a, [rows, col])
                b = plsc.load_gather(buf_b, [rows, col])
                acc = acc + a * b
            out_v[pl.ds(h * _HALF + rb, _LANES)] = acc
            return carry

        lax.fori_loop(0, _HALF // _LANES, body, 0)

    half_body(0)
    half_body(1)

    pltpu.sync_copy(out_v, out_hbm.at[pl.ds(base, _BPW)])


def kernel(i_indices, j_indices, wi, wj, bi, bj):
    ii = i_indices.astype(jnp.int32)
    jj = j_indices.astype(jnp.int32)
    arange = jnp.arange(BATCH, dtype=jnp.int32)
    pi = jnp.argsort(ii)
    si = ii[pi]
    rank_i = jnp.zeros((BATCH,), jnp.int32).at[pi].set(arange)
    pj = jnp.argsort(jj)
    sj = jj[pj]
    rank_j = jnp.zeros((BATCH,), jnp.int32).at[pj].set(arange)

    rows_i, rows_j = _gather_sc(wi.T, wj.T, si, sj)
    return _dot_sc(
        rows_i,
        rows_j,
        bi.reshape(VOCAB),
        bj.reshape(VOCAB),
        ii,
        jj,
        rank_i,
        rank_j,
    )
